# Initial kernel scaffold; baseline (speedup 1.0000x reference)
import jax
import jax.numpy as jnp
from jax.experimental import pallas as pl


def kernel(input_ids, position_ids, token_embedding, position_embedding):
    raise NotImplementedError("write your pallas kernel here")


# speedup vs baseline: 1.4116x; 1.4116x over previous
import functools

import jax
import jax.numpy as jnp
from jax import lax
from jax.experimental import pallas as pl
from jax.experimental.pallas import tpu as pltpu
from jax.experimental.pallas import tpu_sc as plsc

VOCAB = 49408
NPOS = 77
D = 768
ROWS_TOTAL = 4096 * 77
NC, NS, L = 2, 16, 16
NWORK = NC * NS
ROWS_PER_W = ROWS_TOTAL // NWORK
K = 32
NCH = ROWS_PER_W // K

_mesh = plsc.VectorSubcoreMesh(core_axis_name="c", subcore_axis_name="s")


@functools.partial(
    pl.kernel,
    mesh=_mesh,
    out_type=jax.ShapeDtypeStruct((ROWS_TOTAL, D), jnp.float32),
    scratch_types=[
        pltpu.VMEM((K,), jnp.int32),
        pltpu.VMEM((K,), jnp.int32),
        pltpu.VMEM((K, D), jnp.float32),
        pltpu.VMEM((K, D), jnp.float32),
        pltpu.SemaphoreType.DMA,
        pltpu.SemaphoreType.DMA,
    ],
)
def _embed_kernel(tok_hbm, pos_hbm, tid_hbm, pid_hbm, out_hbm,
                  tidx_v, pidx_v, tbuf_v, pbuf_v, sem_t, sem_p):
    wid = lax.axis_index("s") * NC + lax.axis_index("c")
    base_w = wid * ROWS_PER_W

    def chunk(g, carry):
        base = base_w + g * K
        pltpu.sync_copy(tid_hbm.at[pl.ds(base, K)], tidx_v)
        pltpu.sync_copy(pid_hbm.at[pl.ds(base, K)], pidx_v)
        ct = pltpu.async_copy(tok_hbm.at[tidx_v], tbuf_v, sem_t)
        cp = pltpu.async_copy(pos_hbm.at[pidx_v], pbuf_v, sem_p)
        ct.wait()
        cp.wait()

        def row(r, carry2):
            trow = tbuf_v.at[r]
            prow = pbuf_v.at[r]
            for j in range(D // L):
                sl = pl.ds(j * L, L)
                trow[sl] = trow[sl] + prow[sl]
            return carry2

        lax.fori_loop(0, K, row, 0)
        pltpu.sync_copy(tbuf_v, out_hbm.at[pl.ds(base, K)])
        return carry

    lax.fori_loop(0, NCH, chunk, 0)


def kernel(input_ids, position_ids, token_embedding, position_embedding):
    tid = input_ids.reshape(-1).astype(jnp.int32)
    pid = position_ids.reshape(-1).astype(jnp.int32)
    out = _embed_kernel(token_embedding, position_embedding, tid, pid)
    return out.reshape(input_ids.shape + (D,))
